# baseline (speedup 1.0000x reference)
import jax
import jax.numpy as jnp
from jax.experimental import pallas as pl
from jax.experimental.pallas import tpu as pltpu


def _copy_kernel(in_hbm, out_hbm, buf0, buf1, si0, si1, so0, so1):
    bufs = (buf0, buf1)
    in_sems = (si0, si1)
    out_sems = (so0, so1)
    nb = in_hbm.shape[0]

    def in_copy(b):
        return pltpu.make_async_copy(in_hbm.at[b], bufs[b % 2], in_sems[b % 2])

    def out_copy(b):
        return pltpu.make_async_copy(bufs[b % 2], out_hbm.at[b], out_sems[b % 2])

    in_copy(0).start()
    in_copy(1).start()
    for b in range(nb):
        in_copy(b).wait()
        out_copy(b).start()
        if b + 2 < nb:
            out_copy(b).wait()
            in_copy(b + 2).start()
    for b in range(max(nb - 2, 0), nb):
        out_copy(b).wait()


def kernel(inputs_embeds, embed_table):
    del embed_table
    b, s, h = inputs_embeds.shape
    x = inputs_embeds.transpose(0, 2, 1)
    chunk = pltpu.VMEM((h, s), inputs_embeds.dtype)
    sem = pltpu.SemaphoreType.DMA
    out = pl.pallas_call(
        _copy_kernel,
        in_specs=[pl.BlockSpec(memory_space=pl.ANY)],
        out_specs=pl.BlockSpec(memory_space=pl.ANY),
        out_shape=jax.ShapeDtypeStruct((b, h, s), inputs_embeds.dtype),
        scratch_shapes=[chunk, chunk, sem, sem, sem, sem],
    )(x)
    return out.transpose(0, 2, 1)
